# baseline (speedup 1.0000x reference)
import functools

import jax
import jax.numpy as jnp
from jax import lax
from jax.experimental import pallas as pl
from jax.experimental.pallas import tpu as pltpu
from jax.experimental.pallas import tpu_sc as plsc

_NC = 2
_NS = 16
_NW = _NC * _NS
_L = 16
_R = 256
_RSH = 8
_NSAMP = 512


@functools.lru_cache(maxsize=None)
def _build(N, D, B):
    NQ = D // _L
    SPW = B // _NW
    SS = N // _NSAMP
    WLEN = ((SS + 14 + 15) // 16) * 16
    NWV = WLEN // _L
    assert N % _NSAMP == 0 and N % 8 == 0 and WLEN >= SS + 14
    assert (N - WLEN) % 8 == 0 and (N - _R) % 8 == 0

    mesh = plsc.VectorSubcoreMesh(
        core_axis_name="c", subcore_axis_name="s",
        num_cores=_NC, num_subcores=_NS)

    def lanesum(v):
        for k in (1, 2, 4, 8):
            v = v + jnp.take(v, lax.iota(jnp.int32, _L) ^ k)
        return v[0]

    @functools.partial(
        pl.kernel,
        out_type=jax.ShapeDtypeStruct((B, D), jnp.float32),
        mesh=mesh,
        scratch_types=[
            pltpu.VMEM((_NSAMP,), jnp.int32),
            pltpu.VMEM(((SPW + 1) * WLEN,), jnp.int32),
            pltpu.VMEM((_R, D), jnp.float32),
            pltpu.VMEM((_R, D), jnp.float32),
            pltpu.VMEM((SPW, D), jnp.float32),
            pltpu.SMEM((SPW + 8,), jnp.int32),
            pltpu.SemaphoreType.DMA,
            pltpu.SemaphoreType.DMA,
        ],
    )
    def seg_max(feat_hbm, ids_hbm, out_hbm,
                samp_v, win_v, buf0, buf1, outb, bnds_s, sem0, sem1):
        wid = lax.axis_index("s") * _NC + lax.axis_index("c")
        seg0 = wid * SPW

        gd = []
        for part in range(_NSAMP // _L):
            idx = (lax.iota(jnp.int32, _L) + part * _L) * SS
            gd.append(pltpu.async_copy(
                ids_hbm.at[idx], samp_v.at[pl.ds(part * _L, _L)], sem1))
        for d in gd:
            d.wait()

        def count_neg(ref_slice, s_val, nv):
            def cstep(m, acc):
                base = pl.multiple_of(m * (4 * _L), 8)
                for u in range(4):
                    acc = acc + (
                        (ref_slice(base + u * _L) - s_val) >> 31)
                return acc

            return lax.fori_loop(
                0, nv // 4, cstep, jnp.zeros((_L,), jnp.int32))

        wd = []
        As = []
        for jj in range(SPW + 1):
            s_val = seg0 + jj
            acc = count_neg(
                lambda o: samp_v[pl.ds(o, _L)], s_val, _NSAMP // _L)
            j_s = lanesum(acc) * -1
            a0 = (j_s - 1) * SS + 1
            a = jnp.maximum(a0 - 7, 0) & -8
            a = pl.multiple_of(jnp.minimum(a, N - WLEN), 8)
            As.append(a)
            wd.append(pltpu.async_copy(
                ids_hbm.at[pl.ds(a, WLEN)],
                win_v.at[pl.ds(jj * WLEN, WLEN)], sem1))
        ps = []
        for jj in range(SPW + 1):
            wd[jj].wait()
            s_val = seg0 + jj
            acc = count_neg(
                lambda o, _jj=jj: win_v[pl.ds(_jj * WLEN + o, _L)],
                s_val, NWV)
            p = As[jj] - lanesum(acc)
            bnds_s[jj] = p
            ps.append(p)

        iot = lax.iota(jnp.int32, _L)
        vstarts = jnp.zeros((_L,), jnp.int32)
        vends = jnp.zeros((_L,), jnp.int32)
        for j in range(SPW):
            unit = ((iot ^ j) - 1) >> 31
            vstarts = vstarts - unit * ps[j]
            vends = vends - unit * ps[j + 1]

        ninf = jnp.full((_L,), -jnp.inf, jnp.float32)
        for j in range(SPW):
            for q in range(NQ):
                outb[j, q * _L:(q + 1) * _L] = ninf

        w_lo = bnds_s[0]
        w_hi = bnds_s[SPW]
        ws8 = w_lo & -8
        tw = jnp.where(w_hi > w_lo, (w_hi - ws8 + (_R - 1)) >> _RSH, 0)
        npairs = (tw + 1) >> 1

        def srcof(t):
            return pl.multiple_of(jnp.minimum(ws8 + t * _R, N - _R), 8)

        def start(t, buf, sem):
            pltpu.async_copy(feat_hbm.at[pl.ds(srcof(t), _R)], buf, sem)

        def waitbuf(buf, sem):
            pltpu.make_async_copy(
                feat_hbm.at[pl.ds(0, _R)], buf, sem).wait()

        def process(t, buf):
            tl = srcof(t)
            th = tl + _R
            jb = lanesum((vends - (tl + 1)) >> 31) * -1
            je = lanesum((vstarts - th) >> 31) * -1

            def seg_inner(j, c):
                r_lo = bnds_s[j]
                r_hi = bnds_s[j + 1]
                lo = jnp.maximum(r_lo - tl, 0)
                hi = jnp.minimum(r_hi - tl, _R)

                @pl.when(hi > lo)
                def _():
                    accs = tuple(
                        outb[j, q * _L:(q + 1) * _L] for q in range(NQ))
                    n8 = (hi - lo) >> 4

                    def row8(i, a):
                        base = lo + i * 16
                        for u in range(16):
                            a = tuple(
                                jnp.maximum(
                                    a[q],
                                    buf[base + u, q * _L:(q + 1) * _L])
                                for q in range(NQ))
                        return a

                    accs = lax.fori_loop(0, n8, row8, accs)

                    def row1(i, a):
                        return tuple(
                            jnp.maximum(a[q], buf[i, q * _L:(q + 1) * _L])
                            for q in range(NQ))

                    accs = lax.fori_loop(lo + (n8 << 4), hi, row1, accs)
                    for q in range(NQ):
                        outb[j, q * _L:(q + 1) * _L] = accs[q]

                return c

            lax.fori_loop(jb, je, seg_inner, 0)

        start(0, buf0, sem0)

        def pair(k, c):
            t0 = k * 2
            start(t0 + 1, buf1, sem1)
            waitbuf(buf0, sem0)
            process(t0, buf0)
            start(t0 + 2, buf0, sem0)
            waitbuf(buf1, sem1)
            process(t0 + 1, buf1)
            return c

        lax.fori_loop(0, npairs, pair, 0)
        waitbuf(buf0, sem0)
        pltpu.sync_copy(outb, out_hbm.at[pl.ds(seg0, SPW)])

    return seg_max


def kernel(feat, segment_ids, num_segments):
    N, D = feat.shape
    B = 512
    ids = segment_ids.astype(jnp.int32)
    return _build(int(N), int(D), B)(feat, ids)


# speedup vs baseline: 1.1811x; 1.1811x over previous
import functools

import jax
import jax.numpy as jnp
from jax import lax
from jax.experimental import pallas as pl
from jax.experimental.pallas import tpu as pltpu
from jax.experimental.pallas import tpu_sc as plsc

_NC = 2
_NS = 16
_NW = _NC * _NS
_L = 16
_R = 256
_RSH = 8
_NSAMP = 512


@functools.lru_cache(maxsize=None)
def _build(N, D, B):
    NQ = D // _L
    SPW = B // _NW
    SS = N // _NSAMP
    WLEN = ((SS + 14 + 15) // 16) * 16
    NWV = WLEN // _L
    assert N % _NSAMP == 0 and N % 8 == 0 and WLEN >= SS + 14
    assert (N - WLEN) % 8 == 0 and (N - _R) % 8 == 0

    mesh = plsc.VectorSubcoreMesh(
        core_axis_name="c", subcore_axis_name="s",
        num_cores=_NC, num_subcores=_NS)

    def lanesum(v):
        for k in (1, 2, 4, 8):
            v = v + jnp.take(v, lax.iota(jnp.int32, _L) ^ k)
        return v[0]

    @functools.partial(
        pl.kernel,
        out_type=jax.ShapeDtypeStruct((B, D), jnp.float32),
        mesh=mesh,
        scratch_types=[
            pltpu.VMEM((_NSAMP,), jnp.int32),
            pltpu.VMEM(((SPW + 1) * WLEN,), jnp.int32),
            pltpu.VMEM((_R, D), jnp.float32),
            pltpu.VMEM((_R, D), jnp.float32),
            pltpu.VMEM((SPW, D), jnp.float32),
            pltpu.SMEM((SPW + 8,), jnp.int32),
            pltpu.SemaphoreType.DMA,
            pltpu.SemaphoreType.DMA,
        ],
    )
    def seg_max(feat_hbm, ids_hbm, out_hbm,
                samp_v, win_v, buf0, buf1, outb, bnds_s, sem0, sem1):
        wid = lax.axis_index("s") * _NC + lax.axis_index("c")
        seg0 = wid * SPW

        gd = []
        for part in range(_NSAMP // _L):
            idx = (lax.iota(jnp.int32, _L) + part * _L) * SS
            gd.append(pltpu.async_copy(
                ids_hbm.at[idx], samp_v.at[pl.ds(part * _L, _L)], sem1))
        for d in gd:
            d.wait()

        def count_neg(ref_slice, s_val, nv):
            def cstep(m, acc):
                base = pl.multiple_of(m * (4 * _L), 8)
                for u in range(4):
                    acc = acc + (
                        (ref_slice(base + u * _L) - s_val) >> 31)
                return acc

            return lax.fori_loop(
                0, nv // 4, cstep, jnp.zeros((_L,), jnp.int32))

        wd = []
        As = []
        for jj in range(SPW + 1):
            s_val = seg0 + jj
            acc = count_neg(
                lambda o: samp_v[pl.ds(o, _L)], s_val, _NSAMP // _L)
            j_s = lanesum(acc) * -1
            a0 = (j_s - 1) * SS + 1
            a = jnp.maximum(a0 - 7, 0) & -8
            a = pl.multiple_of(jnp.minimum(a, N - WLEN), 8)
            As.append(a)
            wd.append(pltpu.async_copy(
                ids_hbm.at[pl.ds(a, WLEN)],
                win_v.at[pl.ds(jj * WLEN, WLEN)], sem1))
        ps = []
        for jj in range(SPW + 1):
            wd[jj].wait()
            s_val = seg0 + jj
            acc = count_neg(
                lambda o, _jj=jj: win_v[pl.ds(_jj * WLEN + o, _L)],
                s_val, NWV)
            p = As[jj] - lanesum(acc)
            bnds_s[jj] = p
            ps.append(p)

        iot = lax.iota(jnp.int32, _L)
        vstarts = jnp.zeros((_L,), jnp.int32)
        vends = jnp.zeros((_L,), jnp.int32)
        for j in range(SPW):
            unit = ((iot ^ j) - 1) >> 31
            vstarts = vstarts - unit * ps[j]
            vends = vends - unit * ps[j + 1]

        ninf = jnp.full((_L,), -jnp.inf, jnp.float32)
        for j in range(SPW):
            for q in range(NQ):
                outb[j, q * _L:(q + 1) * _L] = ninf

        w_lo = bnds_s[0]
        w_hi = bnds_s[SPW]
        ws8 = w_lo & -8
        tw = jnp.where(w_hi > w_lo, (w_hi - ws8 + (_R - 1)) >> _RSH, 0)
        npairs = (tw + 1) >> 1

        def srcof(t):
            return pl.multiple_of(jnp.minimum(ws8 + t * _R, N - _R), 8)

        def start(t, buf, sem):
            pltpu.async_copy(feat_hbm.at[pl.ds(srcof(t), _R)], buf, sem)

        def waitbuf(buf, sem):
            pltpu.make_async_copy(
                feat_hbm.at[pl.ds(0, _R)], buf, sem).wait()

        def process(t, buf):
            tl = srcof(t)
            th = tl + _R
            jb = lanesum((vends - (tl + 1)) >> 31) * -1
            je = lanesum((vstarts - th) >> 31) * -1

            def seg_inner(j, c):
                r_lo = bnds_s[j]
                r_hi = bnds_s[j + 1]
                lo = jnp.maximum(r_lo - tl, 0)
                hi = jnp.minimum(r_hi - tl, _R)

                @pl.when(hi > lo)
                def _():
                    accs = tuple(
                        outb[j, q * _L:(q + 1) * _L] for q in range(NQ))
                    n8 = (hi - lo) >> 3

                    def row8(i, a):
                        base = lo + i * 8
                        for u in range(8):
                            a = tuple(
                                jnp.maximum(
                                    a[q],
                                    buf[base + u, q * _L:(q + 1) * _L])
                                for q in range(NQ))
                        return a

                    accs = lax.fori_loop(0, n8, row8, accs)

                    def row1(i, a):
                        return tuple(
                            jnp.maximum(a[q], buf[i, q * _L:(q + 1) * _L])
                            for q in range(NQ))

                    accs = lax.fori_loop(lo + (n8 << 3), hi, row1, accs)
                    for q in range(NQ):
                        outb[j, q * _L:(q + 1) * _L] = accs[q]

                return c

            lax.fori_loop(jb, je, seg_inner, 0)

        start(0, buf0, sem0)

        def pair(k, c):
            t0 = k * 2
            start(t0 + 1, buf1, sem1)
            waitbuf(buf0, sem0)
            process(t0, buf0)
            start(t0 + 2, buf0, sem0)
            waitbuf(buf1, sem1)
            process(t0 + 1, buf1)
            return c

        lax.fori_loop(0, npairs, pair, 0)
        waitbuf(buf0, sem0)
        pltpu.sync_copy(outb, out_hbm.at[pl.ds(seg0, SPW)])

    return seg_max


def kernel(feat, segment_ids, num_segments):
    N, D = feat.shape
    B = 512
    ids = segment_ids.astype(jnp.int32)
    return _build(int(N), int(D), B)(feat, ids)
